# baseline (speedup 1.0000x reference)
import functools

import numpy as np
import jax
import jax.numpy as jnp
from jax import lax
from jax.experimental import pallas as pl
from jax.experimental.pallas import tpu as pltpu
from jax.experimental.pallas import tpu_sc as plsc

_F32 = jnp.float32
_N = 10000
_E = 320000
_B = 16
_T = 1280
_C = 400
_EPS = 1e-5
_HI = lax.Precision.HIGHEST


def _gn(t, mmat, gamma, beta, prec=None):
    mean = jnp.dot(t, mmat, preferred_element_type=_F32, precision=prec)
    d = t - mean
    var = jnp.dot(d * d, mmat, preferred_element_type=_F32, precision=prec)
    return d * lax.rsqrt(var + _EPS) * gamma + beta


def _enc_body(xin, w1, w2, w3, wa, wb, vecs, m64, ha_ref, hb_ref):
    e = pl.program_id(0)
    x = xin[0]
    b1 = vecs[0, 0]; g1 = vecs[0, 1]; be1 = vecs[0, 2]
    b2 = vecs[0, 3]; g2 = vecs[0, 4]; be2 = vecs[0, 5]
    b3 = vecs[0, 6]
    h = jnp.dot(x, w1[0], preferred_element_type=_F32, precision=_HI) + b1
    h = jax.nn.relu(_gn(h, m64[...], g1, be1, _HI))
    h = jnp.dot(h, w2[0], preferred_element_type=_F32, precision=_HI) + b2
    h = jax.nn.relu(_gn(h, m64[...], g2, be2, _HI))
    h = jnp.dot(h, w3[0], preferred_element_type=_F32, precision=_HI) + b3
    ca = jnp.dot(h, wa[0], preferred_element_type=_F32, precision=_HI)
    cb = jnp.dot(h, wb[0], preferred_element_type=_F32, precision=_HI)

    @pl.when(e == 0)
    def _():
        ha_ref[...] = ca
        hb_ref[...] = cb

    @pl.when(e != 0)
    def _():
        ha_ref[...] = ha_ref[...] + ca
        hb_ref[...] = hb_ref[...] + cb


def _encoders(xin, w1, w2, w3, wa, wb, vecs, m64):
    n = xin.shape[1]
    return pl.pallas_call(
        _enc_body,
        grid=(2,),
        in_specs=[
            pl.BlockSpec((1, n, 4), lambda e: (e, 0, 0)),
            pl.BlockSpec((1, 4, 64), lambda e: (e, 0, 0)),
            pl.BlockSpec((1, 64, 64), lambda e: (e, 0, 0)),
            pl.BlockSpec((1, 64, 64), lambda e: (e, 0, 0)),
            pl.BlockSpec((1, 64, 128), lambda e: (e, 0, 0)),
            pl.BlockSpec((1, 64, 128), lambda e: (e, 0, 0)),
            pl.BlockSpec((1, 8, 64), lambda e: (e, 0, 0)),
            pl.BlockSpec((64, 64), lambda e: (0, 0)),
        ],
        out_specs=[
            pl.BlockSpec((n, 128), lambda e: (0, 0)),
            pl.BlockSpec((n, 128), lambda e: (0, 0)),
        ],
        out_shape=[
            jax.ShapeDtypeStruct((n, 128), _F32),
            jax.ShapeDtypeStruct((n, 128), _F32),
        ],
    )(xin, w1, w2, w3, wa, wb, vecs, m64)


def _sc_gather(ha, hb, dst, src):
    info = plsc.get_sparse_core_info()
    nw = info.num_cores * info.num_subcores
    e = dst.shape[0]
    per_w = e // nw
    mesh = plsc.VectorSubcoreMesh(core_axis_name="c", subcore_axis_name="s")

    @functools.partial(
        pl.kernel,
        mesh=mesh,
        out_type=[
            jax.ShapeDtypeStruct((e, 128), _F32),
            jax.ShapeDtypeStruct((e, 128), _F32),
        ],
        scratch_types=[
            pltpu.VMEM((_C,), jnp.int32),
            pltpu.VMEM((_C, 128), _F32),
            pltpu.VMEM((_C,), jnp.int32),
            pltpu.VMEM((_C, 128), _F32),
            pltpu.SemaphoreType.DMA,
            pltpu.SemaphoreType.DMA,
        ],
    )
    def k(ha_hbm, hb_hbm, dst_hbm, src_hbm, ai_hbm, bj_hbm,
          idxa_v, rowsa_v, idxb_v, rowsb_v, sema, semb):
        wid = lax.axis_index("s") * info.num_cores + lax.axis_index("c")
        base = wid * per_w

        def body(ci, _):
            off = base + ci * _C
            pltpu.sync_copy(dst_hbm.at[pl.ds(off, _C)], idxa_v)
            pltpu.sync_copy(src_hbm.at[pl.ds(off, _C)], idxb_v)
            cpa = pltpu.async_copy(ha_hbm.at[idxa_v], rowsa_v, sema)
            cpb = pltpu.async_copy(hb_hbm.at[idxb_v], rowsb_v, semb)
            cpa.wait()
            pltpu.sync_copy(rowsa_v, ai_hbm.at[pl.ds(off, _C)])
            cpb.wait()
            pltpu.sync_copy(rowsb_v, bj_hbm.at[pl.ds(off, _C)])
            return 0

        lax.fori_loop(0, per_w // _C, body, 0)

    return k(ha, hb, dst, src)


def _edge_body(ai, bj, w2, w3, vecs, m128, mt_ref):
    g1 = vecs[0]; be1 = vecs[1]; g2 = vecs[2]
    be2 = vecs[3]; g3 = vecs[4]; be3 = vecs[5]
    pre = ai[...] + bj[...]
    m = jax.nn.relu(_gn(pre, m128[...], g1, be1))
    m = jnp.dot(m, w2[...], preferred_element_type=_F32)
    m = jax.nn.relu(_gn(m, m128[...], g2, be2))
    m = jnp.dot(m, w3[...], preferred_element_type=_F32)
    m = jax.nn.relu(_gn(m, m128[...], g3, be3))
    mt_ref[...] = m.T


def _edge_mlp(ai, bj, w2, w3, vecs, m128, hb_):
    nt = _E // 2 // _T
    t0 = hb_ * nt
    return pl.pallas_call(
        _edge_body,
        grid=(nt,),
        in_specs=[
            pl.BlockSpec((_T, 128), lambda t: (t0 + t, 0)),
            pl.BlockSpec((_T, 128), lambda t: (t0 + t, 0)),
            pl.BlockSpec((128, 128), lambda t: (0, 0)),
            pl.BlockSpec((128, 128), lambda t: (0, 0)),
            pl.BlockSpec((8, 128), lambda t: (0, 0)),
            pl.BlockSpec((128, 128), lambda t: (0, 0)),
        ],
        out_specs=pl.BlockSpec((128, _T), lambda t: (0, t)),
        out_shape=jax.ShapeDtypeStruct((128, _E // 2), _F32),
    )(ai, bj, w2, w3, vecs, m128)


_C2 = 640
_CPW = 8


def _sc_segment_max(mt, dst, e_base):
    info = plsc.get_sparse_core_info()
    nw = info.num_cores * info.num_subcores
    assert nw * _CPW == 2 * 128
    eh = mt.shape[1] // 2
    mesh = plsc.VectorSubcoreMesh(core_axis_name="c", subcore_axis_name="s")

    @functools.partial(
        pl.kernel,
        mesh=mesh,
        compiler_params=pltpu.CompilerParams(needs_layout_passes=False),
        out_type=jax.ShapeDtypeStruct((2, 128 * _N), _F32),
        scratch_types=[
            pltpu.VMEM((_CPW * _N,), _F32),
            pltpu.VMEM((_CPW * _C2,), _F32),
            pltpu.VMEM((_CPW * _C2,), _F32),
            pltpu.VMEM((_C2,), jnp.int32),
            pltpu.VMEM((_C2,), jnp.int32),
            pltpu.VMEM((_N,), jnp.int32),
            pltpu.SemaphoreType.DMA,
        ],
    )
    def k(mt_hbm, dst_hbm, aggt_hbm, acc_v, mtv0_v, mtv1_v, idx0_v, idx1_v,
          test_v, sem):
        wid = lax.axis_index("s") * info.num_cores + lax.axis_index("c")
        half = wid % 2
        c0 = (wid // 2) * _CPW
        iota16 = lax.iota(jnp.int32, 16)
        nch = eh // _C2

        zeros = jnp.zeros((16,), _F32)
        def z_body(i, _):
            acc_v[pl.ds(i * 16, 16)] = zeros
            return 0
        lax.fori_loop(0, _CPW * _N // 16, z_body, 0)

        def dma_pairs(ci, ib, mb):
            el = half * eh + ci * _C2
            pairs = [(dst_hbm.at[pl.ds(e_base + el, _C2)], ib)]
            for c in range(_CPW):
                pairs.append((mt_hbm.at[c0 + c, pl.ds(el, _C2)],
                              mb.at[pl.ds(c * _C2, _C2)]))
            return pairs

        def start(ci, ib, mb):
            @pl.when(ci < nch)
            def _():
                for s_, d_ in dma_pairs(ci, ib, mb):
                    pltpu.async_copy(s_, d_, sem)

        def drain(ci, ib, mb):
            for s_, d_ in dma_pairs(ci, ib, mb):
                pltpu.make_async_copy(s_, d_, sem).wait()

        def process(idx_v, mtv_v):
            def grp_body(g, _):
                d0 = idx_v[pl.ds(g * 32, 16)]
                d1 = idx_v[pl.ds(g * 32 + 16, 16)]
                plsc.store_scatter(test_v, [d0], iota16)
                plsc.store_scatter(test_v, [d1], iota16 + 16)
                b0 = plsc.load_gather(test_v, [d0])
                b1 = plsc.load_gather(test_v, [d1])
                dup = jnp.any((b0 != iota16) | (b1 != (iota16 + 16)))
                vals0 = [mtv_v[pl.ds(c * _C2 + g * 32, 16)]
                         for c in range(_CPW)]
                vals1 = [mtv_v[pl.ds(c * _C2 + g * 32 + 16, 16)]
                         for c in range(_CPW)]

                def fast(_a):
                    for c in range(_CPW):
                        f0 = d0 + c * _N
                        f1 = d1 + c * _N
                        o0 = plsc.load_gather(acc_v, [f0])
                        o1 = plsc.load_gather(acc_v, [f1])
                        plsc.store_scatter(acc_v, [f0],
                                           jnp.maximum(o0, vals0[c]))
                        plsc.store_scatter(acc_v, [f1],
                                           jnp.maximum(o1, vals1[c]))
                    return 0

                def slow(_a):
                    for dv, vv in ((d0, vals0), (d1, vals1)):
                        for c in range(_CPW):
                            fidx = dv + c * _N
                            vals = vv[c]
                            old = plsc.load_gather(acc_v, [fidx])
                            plsc.store_scatter(acc_v, [fidx],
                                               jnp.maximum(old, vals))

                            def cond(_c):
                                bk = plsc.load_gather(acc_v, [fidx])
                                return jnp.any(bk < vals)

                            def fix(_c):
                                bk = plsc.load_gather(acc_v, [fidx])
                                plsc.store_scatter(acc_v, [fidx],
                                                   jnp.maximum(bk, vals),
                                                   mask=bk < vals)
                                return 0

                            lax.while_loop(cond, fix, 0)
                    return 0

                lax.cond(dup, slow, fast, 0)
                return 0

            lax.fori_loop(0, _C2 // 32, grp_body, 0)

        start(0, idx0_v, mtv0_v)
        start(1, idx1_v, mtv1_v)

        def pair_body(i, _):
            k0 = 2 * i
            drain(k0, idx0_v, mtv0_v)
            process(idx0_v, mtv0_v)
            start(k0 + 2, idx0_v, mtv0_v)
            drain(k0 + 1, idx1_v, mtv1_v)
            process(idx1_v, mtv1_v)
            start(k0 + 3, idx1_v, mtv1_v)
            return 0

        lax.fori_loop(0, nch // 2, pair_body, 0)
        if nch % 2:
            drain(nch - 1, idx0_v, mtv0_v)
            process(idx0_v, mtv0_v)
        pltpu.sync_copy(acc_v, aggt_hbm.at[half, pl.ds(c0 * _N, _CPW * _N)])

    return k(mt, dst)


def _gn_t(t, mmat, gamma, beta):
    mean = jnp.dot(mmat, t, preferred_element_type=_F32, precision=_HI)
    d = t - mean
    var = jnp.dot(mmat, d * d, preferred_element_type=_F32, precision=_HI)
    return d * lax.rsqrt(var + _EPS) * gamma + beta


def _tail_body(p1, p2, bmaskt, ew1t, ew2t, evecst, fw1t, fw2t, fw3t, fvecst,
               m128, out_ref):
    aggt = jnp.maximum(jnp.maximum(p1[0], p1[1]),
                       jnp.maximum(p2[0], p2[1]))
    h = jnp.dot(ew1t[...], aggt, preferred_element_type=_F32, precision=_HI)
    h = jax.nn.relu(_gn_t(h, m128[...], evecst[:, 0:1], evecst[:, 1:2]))
    h = jnp.dot(ew2t[...], h, preferred_element_type=_F32, precision=_HI)
    h = jax.nn.relu(_gn_t(h, m128[...], evecst[:, 2:3], evecst[:, 3:4]))
    bm = bmaskt[...]
    cols = []
    for b in range(_B):
        mb = bm[b:b + 1, :]
        cols.append(jnp.max(h * mb, axis=1, keepdims=True))
    gft = jnp.concatenate(cols, axis=1)
    o = jax.nn.relu(jnp.dot(fw1t[...], gft, preferred_element_type=_F32, precision=_HI)
                    + fvecst[:, 0:1])
    o = jax.nn.relu(jnp.dot(fw2t[...], o, preferred_element_type=_F32, precision=_HI)
                    + fvecst[:, 1:2])
    ot = (jnp.dot(fw3t[...], o, preferred_element_type=_F32, precision=_HI)
          + fvecst[:6, 2:3])
    out_ref[...] = ot.T


def _tail(p1, p2, bmaskt, ew1t, ew2t, evecst, fw1t, fw2t, fw3t, fvecst, m128):
    return pl.pallas_call(
        _tail_body,
        out_shape=jax.ShapeDtypeStruct((_B, 6), _F32),
    )(p1, p2, bmaskt, ew1t, ew2t, evecst, fw1t, fw2t, fw3t, fvecst, m128)


def kernel(x, pos, edge_index, batch, batch_size, params):
    p = params
    src = edge_index[0]
    dst = edge_index[1]

    m64 = jnp.asarray(np.kron(np.eye(4), np.ones((16, 16)) / 16.0), _F32)
    m128 = jnp.asarray(np.kron(np.eye(8), np.ones((16, 16)) / 16.0), _F32)

    w1 = p['loc_w1']
    wa = w1[:128] - w1[128:]
    wb = w1[128:]

    xin = jnp.stack([x, pos])
    w1s = jnp.stack([p['enc_w1'], p['pos_w1']])
    w2s = jnp.stack([p['enc_w2'], p['pos_w2']])
    w3s = jnp.stack([p['enc_w3'], p['pos_w3']])
    was = jnp.stack([wa[:64], wa[64:]])
    wbs = jnp.stack([wb[:64], wb[64:]])
    zpad = jnp.zeros((64,), _F32)
    evec = lambda pre: jnp.stack([
        p[pre + '_b1'], p[pre + '_g1'], p[pre + '_be1'],
        p[pre + '_b2'], p[pre + '_g2'], p[pre + '_be2'],
        p[pre + '_b3'], zpad])
    vecs64 = jnp.stack([evec('enc'), evec('pos')])

    ha, hb = _encoders(xin, w1s, w2s, w3s, was, wbs, vecs64, m64)
    ai, bj = _sc_gather(ha, hb, dst, src)

    zpad128 = jnp.zeros((128,), _F32)
    locv = jnp.stack([p['loc_g1'], p['loc_be1'], p['loc_g2'], p['loc_be2'],
                      p['loc_g3'], p['loc_be3'], zpad128, zpad128])
    mt1 = _edge_mlp(ai, bj, p['loc_w2'], p['loc_w3'], locv, m128, 0)
    p1 = _sc_segment_max(mt1, dst, 0)
    mt2 = _edge_mlp(ai, bj, p['loc_w2'], p['loc_w3'], locv, m128, 1)
    p2 = _sc_segment_max(mt2, dst, _E // 2)
    p1 = p1.reshape(2, 128, _N)
    p2 = p2.reshape(2, 128, _N)

    bmaskt = (batch[None, :] == jnp.arange(_B, dtype=jnp.int32)[:, None]
              ).astype(_F32)
    evecst = jnp.stack([p['e2_g1'], p['e2_be1'], p['e2_g2'], p['e2_be2']],
                       axis=1)
    fb3 = jnp.concatenate([p['fc_b3'], jnp.zeros((122,), _F32)])
    fvecst = jnp.stack([p['fc_b1'], p['fc_b2'], fb3], axis=1)
    logits = _tail(p1, p2, bmaskt, p['e2_w1'].T, p['e2_w2'].T, evecst,
                   p['fc_w1'].T, p['fc_w2'].T, p['fc_w3'].T, fvecst, m128)
    return logits


# speedup vs baseline: 1.1435x; 1.1435x over previous
import functools

import numpy as np
import jax
import jax.numpy as jnp
from jax import lax
from jax.experimental import pallas as pl
from jax.experimental.pallas import tpu as pltpu
from jax.experimental.pallas import tpu_sc as plsc

_F32 = jnp.float32
_N = 10000
_E = 320000
_B = 16
_T = 1280
_C = 400
_EPS = 1e-5


def _gn(t, mmat, gamma, beta):
    mean = jnp.dot(t, mmat, preferred_element_type=_F32)
    d = t - mean
    var = jnp.dot(d * d, mmat, preferred_element_type=_F32)
    return d * lax.rsqrt(var + _EPS) * gamma + beta


def _enc_body(xin, w1, w2, w3, wa, wb, vecs, m64, ha_ref, hb_ref):
    e = pl.program_id(0)
    x = xin[0]
    b1 = vecs[0, 0]; g1 = vecs[0, 1]; be1 = vecs[0, 2]
    b2 = vecs[0, 3]; g2 = vecs[0, 4]; be2 = vecs[0, 5]
    b3 = vecs[0, 6]
    h = jnp.dot(x, w1[0], preferred_element_type=_F32) + b1
    h = jax.nn.relu(_gn(h, m64[...], g1, be1))
    h = jnp.dot(h, w2[0], preferred_element_type=_F32) + b2
    h = jax.nn.relu(_gn(h, m64[...], g2, be2))
    h = jnp.dot(h, w3[0], preferred_element_type=_F32) + b3
    ca = jnp.dot(h, wa[0], preferred_element_type=_F32)
    cb = jnp.dot(h, wb[0], preferred_element_type=_F32)

    @pl.when(e == 0)
    def _():
        ha_ref[...] = ca
        hb_ref[...] = cb

    @pl.when(e != 0)
    def _():
        ha_ref[...] = ha_ref[...] + ca
        hb_ref[...] = hb_ref[...] + cb


def _encoders(xin, w1, w2, w3, wa, wb, vecs, m64):
    n = xin.shape[1]
    return pl.pallas_call(
        _enc_body,
        grid=(2,),
        in_specs=[
            pl.BlockSpec((1, n, 4), lambda e: (e, 0, 0)),
            pl.BlockSpec((1, 4, 64), lambda e: (e, 0, 0)),
            pl.BlockSpec((1, 64, 64), lambda e: (e, 0, 0)),
            pl.BlockSpec((1, 64, 64), lambda e: (e, 0, 0)),
            pl.BlockSpec((1, 64, 128), lambda e: (e, 0, 0)),
            pl.BlockSpec((1, 64, 128), lambda e: (e, 0, 0)),
            pl.BlockSpec((1, 8, 64), lambda e: (e, 0, 0)),
            pl.BlockSpec((64, 64), lambda e: (0, 0)),
        ],
        out_specs=[
            pl.BlockSpec((n, 128), lambda e: (0, 0)),
            pl.BlockSpec((n, 128), lambda e: (0, 0)),
        ],
        out_shape=[
            jax.ShapeDtypeStruct((n, 128), _F32),
            jax.ShapeDtypeStruct((n, 128), _F32),
        ],
    )(xin, w1, w2, w3, wa, wb, vecs, m64)


def _sc_gather(ha, hb, dst, src):
    info = plsc.get_sparse_core_info()
    nw = info.num_cores * info.num_subcores
    e = dst.shape[0]
    per_w = e // nw
    mesh = plsc.VectorSubcoreMesh(core_axis_name="c", subcore_axis_name="s")

    @functools.partial(
        pl.kernel,
        mesh=mesh,
        out_type=[
            jax.ShapeDtypeStruct((e, 128), _F32),
            jax.ShapeDtypeStruct((e, 128), _F32),
        ],
        scratch_types=[
            pltpu.VMEM((_C,), jnp.int32),
            pltpu.VMEM((_C, 128), _F32),
            pltpu.VMEM((_C,), jnp.int32),
            pltpu.VMEM((_C, 128), _F32),
            pltpu.SemaphoreType.DMA,
            pltpu.SemaphoreType.DMA,
        ],
    )
    def k(ha_hbm, hb_hbm, dst_hbm, src_hbm, ai_hbm, bj_hbm,
          idxa_v, rowsa_v, idxb_v, rowsb_v, sema, semb):
        wid = lax.axis_index("s") * info.num_cores + lax.axis_index("c")
        base = wid * per_w

        def body(ci, _):
            off = base + ci * _C
            pltpu.sync_copy(dst_hbm.at[pl.ds(off, _C)], idxa_v)
            pltpu.sync_copy(src_hbm.at[pl.ds(off, _C)], idxb_v)
            cpa = pltpu.async_copy(ha_hbm.at[idxa_v], rowsa_v, sema)
            cpb = pltpu.async_copy(hb_hbm.at[idxb_v], rowsb_v, semb)
            cpa.wait()
            pltpu.sync_copy(rowsa_v, ai_hbm.at[pl.ds(off, _C)])
            cpb.wait()
            pltpu.sync_copy(rowsb_v, bj_hbm.at[pl.ds(off, _C)])
            return 0

        lax.fori_loop(0, per_w // _C, body, 0)

    return k(ha, hb, dst, src)


def _edge_body(ai, bj, w2, w3, vecs, m128, mt_ref):
    g1 = vecs[0]; be1 = vecs[1]; g2 = vecs[2]
    be2 = vecs[3]; g3 = vecs[4]; be3 = vecs[5]
    pre = ai[...] + bj[...]
    m = jax.nn.relu(_gn(pre, m128[...], g1, be1))
    m = jnp.dot(m, w2[...], preferred_element_type=_F32)
    m = jax.nn.relu(_gn(m, m128[...], g2, be2))
    m = jnp.dot(m, w3[...], preferred_element_type=_F32)
    m = jax.nn.relu(_gn(m, m128[...], g3, be3))
    mt_ref[...] = m.T


def _edge_mlp(ai, bj, w2, w3, vecs, m128, hb_):
    nt = _E // 2 // _T
    t0 = hb_ * nt
    return pl.pallas_call(
        _edge_body,
        grid=(nt,),
        in_specs=[
            pl.BlockSpec((_T, 128), lambda t: (t0 + t, 0)),
            pl.BlockSpec((_T, 128), lambda t: (t0 + t, 0)),
            pl.BlockSpec((128, 128), lambda t: (0, 0)),
            pl.BlockSpec((128, 128), lambda t: (0, 0)),
            pl.BlockSpec((8, 128), lambda t: (0, 0)),
            pl.BlockSpec((128, 128), lambda t: (0, 0)),
        ],
        out_specs=pl.BlockSpec((128, _T), lambda t: (0, t)),
        out_shape=jax.ShapeDtypeStruct((128, _E // 2), _F32),
    )(ai, bj, w2, w3, vecs, m128)


_C2 = 640
_CPW = 8


def _sc_segment_max(mt, dst, e_base):
    info = plsc.get_sparse_core_info()
    nw = info.num_cores * info.num_subcores
    assert nw * _CPW == 2 * 128
    eh = mt.shape[1] // 2
    mesh = plsc.VectorSubcoreMesh(core_axis_name="c", subcore_axis_name="s")

    @functools.partial(
        pl.kernel,
        mesh=mesh,
        compiler_params=pltpu.CompilerParams(needs_layout_passes=False),
        out_type=jax.ShapeDtypeStruct((2, 128 * _N), _F32),
        scratch_types=[
            pltpu.VMEM((_CPW * _N,), _F32),
            pltpu.VMEM((_CPW * _C2,), _F32),
            pltpu.VMEM((_CPW * _C2,), _F32),
            pltpu.VMEM((_C2,), jnp.int32),
            pltpu.VMEM((_C2,), jnp.int32),
            pltpu.VMEM((_N,), jnp.int32),
            pltpu.SemaphoreType.DMA,
        ],
    )
    def k(mt_hbm, dst_hbm, aggt_hbm, acc_v, mtv0_v, mtv1_v, idx0_v, idx1_v,
          test_v, sem):
        wid = lax.axis_index("s") * info.num_cores + lax.axis_index("c")
        half = wid % 2
        c0 = (wid // 2) * _CPW
        iota16 = lax.iota(jnp.int32, 16)
        nch = eh // _C2

        zeros = jnp.zeros((16,), _F32)
        def z_body(i, _):
            acc_v[pl.ds(i * 16, 16)] = zeros
            return 0
        lax.fori_loop(0, _CPW * _N // 16, z_body, 0)

        def dma_pairs(ci, ib, mb):
            el = half * eh + ci * _C2
            pairs = [(dst_hbm.at[pl.ds(e_base + el, _C2)], ib)]
            for c in range(_CPW):
                pairs.append((mt_hbm.at[c0 + c, pl.ds(el, _C2)],
                              mb.at[pl.ds(c * _C2, _C2)]))
            return pairs

        def start(ci, ib, mb):
            @pl.when(ci < nch)
            def _():
                for s_, d_ in dma_pairs(ci, ib, mb):
                    pltpu.async_copy(s_, d_, sem)

        def drain(ci, ib, mb):
            for s_, d_ in dma_pairs(ci, ib, mb):
                pltpu.make_async_copy(s_, d_, sem).wait()

        def process(idx_v, mtv_v):
            def grp_body(g, _):
                d0 = idx_v[pl.ds(g * 32, 16)]
                d1 = idx_v[pl.ds(g * 32 + 16, 16)]
                plsc.store_scatter(test_v, [d0], iota16)
                plsc.store_scatter(test_v, [d1], iota16 + 16)
                b0 = plsc.load_gather(test_v, [d0])
                b1 = plsc.load_gather(test_v, [d1])
                dup = jnp.any((b0 != iota16) | (b1 != (iota16 + 16)))
                vals0 = [mtv_v[pl.ds(c * _C2 + g * 32, 16)]
                         for c in range(_CPW)]
                vals1 = [mtv_v[pl.ds(c * _C2 + g * 32 + 16, 16)]
                         for c in range(_CPW)]

                def fast(_a):
                    for c in range(_CPW):
                        f0 = d0 + c * _N
                        f1 = d1 + c * _N
                        o0 = plsc.load_gather(acc_v, [f0])
                        o1 = plsc.load_gather(acc_v, [f1])
                        plsc.store_scatter(acc_v, [f0],
                                           jnp.maximum(o0, vals0[c]))
                        plsc.store_scatter(acc_v, [f1],
                                           jnp.maximum(o1, vals1[c]))
                    return 0

                def slow(_a):
                    for dv, vv in ((d0, vals0), (d1, vals1)):
                        for c in range(_CPW):
                            fidx = dv + c * _N
                            vals = vv[c]
                            old = plsc.load_gather(acc_v, [fidx])
                            plsc.store_scatter(acc_v, [fidx],
                                               jnp.maximum(old, vals))

                            def cond(_c):
                                bk = plsc.load_gather(acc_v, [fidx])
                                return jnp.any(bk < vals)

                            def fix(_c):
                                bk = plsc.load_gather(acc_v, [fidx])
                                plsc.store_scatter(acc_v, [fidx],
                                                   jnp.maximum(bk, vals),
                                                   mask=bk < vals)
                                return 0

                            lax.while_loop(cond, fix, 0)
                    return 0

                lax.cond(dup, slow, fast, 0)
                return 0

            lax.fori_loop(0, _C2 // 32, grp_body, 0)

        start(0, idx0_v, mtv0_v)
        start(1, idx1_v, mtv1_v)

        def pair_body(i, _):
            k0 = 2 * i
            drain(k0, idx0_v, mtv0_v)
            process(idx0_v, mtv0_v)
            start(k0 + 2, idx0_v, mtv0_v)
            drain(k0 + 1, idx1_v, mtv1_v)
            process(idx1_v, mtv1_v)
            start(k0 + 3, idx1_v, mtv1_v)
            return 0

        lax.fori_loop(0, nch // 2, pair_body, 0)
        if nch % 2:
            drain(nch - 1, idx0_v, mtv0_v)
            process(idx0_v, mtv0_v)
        pltpu.sync_copy(acc_v, aggt_hbm.at[half, pl.ds(c0 * _N, _CPW * _N)])

    return k(mt, dst)


def _gn_t(t, mmat, gamma, beta):
    mean = jnp.dot(mmat, t, preferred_element_type=_F32)
    d = t - mean
    var = jnp.dot(mmat, d * d, preferred_element_type=_F32)
    return d * lax.rsqrt(var + _EPS) * gamma + beta


def _tail_body(p1, p2, bmaskt, ew1t, ew2t, evecst, fw1t, fw2t, fw3t, fvecst,
               m128, out_ref):
    aggt = jnp.maximum(jnp.maximum(p1[0], p1[1]),
                       jnp.maximum(p2[0], p2[1]))
    h = jnp.dot(ew1t[...], aggt, preferred_element_type=_F32)
    h = jax.nn.relu(_gn_t(h, m128[...], evecst[:, 0:1], evecst[:, 1:2]))
    h = jnp.dot(ew2t[...], h, preferred_element_type=_F32)
    h = jax.nn.relu(_gn_t(h, m128[...], evecst[:, 2:3], evecst[:, 3:4]))
    bm = bmaskt[...]
    cols = []
    for b in range(_B):
        mb = bm[b:b + 1, :]
        cols.append(jnp.max(h * mb, axis=1, keepdims=True))
    gft = jnp.concatenate(cols, axis=1)
    o = jax.nn.relu(jnp.dot(fw1t[...], gft, preferred_element_type=_F32)
                    + fvecst[:, 0:1])
    o = jax.nn.relu(jnp.dot(fw2t[...], o, preferred_element_type=_F32)
                    + fvecst[:, 1:2])
    ot = (jnp.dot(fw3t[...], o, preferred_element_type=_F32)
          + fvecst[:6, 2:3])
    out_ref[...] = ot.T


def _tail(p1, p2, bmaskt, ew1t, ew2t, evecst, fw1t, fw2t, fw3t, fvecst, m128):
    return pl.pallas_call(
        _tail_body,
        out_shape=jax.ShapeDtypeStruct((_B, 6), _F32),
    )(p1, p2, bmaskt, ew1t, ew2t, evecst, fw1t, fw2t, fw3t, fvecst, m128)


def kernel(x, pos, edge_index, batch, batch_size, params):
    p = params
    src = edge_index[0]
    dst = edge_index[1]

    m64 = jnp.asarray(np.kron(np.eye(4), np.ones((16, 16)) / 16.0), _F32)
    m128 = jnp.asarray(np.kron(np.eye(8), np.ones((16, 16)) / 16.0), _F32)

    w1 = p['loc_w1']
    wa = w1[:128] - w1[128:]
    wb = w1[128:]

    xin = jnp.stack([x, pos])
    w1s = jnp.stack([p['enc_w1'], p['pos_w1']])
    w2s = jnp.stack([p['enc_w2'], p['pos_w2']])
    w3s = jnp.stack([p['enc_w3'], p['pos_w3']])
    was = jnp.stack([wa[:64], wa[64:]])
    wbs = jnp.stack([wb[:64], wb[64:]])
    zpad = jnp.zeros((64,), _F32)
    evec = lambda pre: jnp.stack([
        p[pre + '_b1'], p[pre + '_g1'], p[pre + '_be1'],
        p[pre + '_b2'], p[pre + '_g2'], p[pre + '_be2'],
        p[pre + '_b3'], zpad])
    vecs64 = jnp.stack([evec('enc'), evec('pos')])

    ha, hb = _encoders(xin, w1s, w2s, w3s, was, wbs, vecs64, m64)
    ai, bj = _sc_gather(ha, hb, dst, src)

    zpad128 = jnp.zeros((128,), _F32)
    locv = jnp.stack([p['loc_g1'], p['loc_be1'], p['loc_g2'], p['loc_be2'],
                      p['loc_g3'], p['loc_be3'], zpad128, zpad128])
    mt1 = _edge_mlp(ai, bj, p['loc_w2'], p['loc_w3'], locv, m128, 0)
    p1 = _sc_segment_max(mt1, dst, 0)
    mt2 = _edge_mlp(ai, bj, p['loc_w2'], p['loc_w3'], locv, m128, 1)
    p2 = _sc_segment_max(mt2, dst, _E // 2)
    p1 = p1.reshape(2, 128, _N)
    p2 = p2.reshape(2, 128, _N)

    bmaskt = (batch[None, :] == jnp.arange(_B, dtype=jnp.int32)[:, None]
              ).astype(_F32)
    evecst = jnp.stack([p['e2_g1'], p['e2_be1'], p['e2_g2'], p['e2_be2']],
                       axis=1)
    fb3 = jnp.concatenate([p['fc_b3'], jnp.zeros((122,), _F32)])
    fvecst = jnp.stack([p['fc_b1'], p['fc_b2'], fb3], axis=1)
    logits = _tail(p1, p2, bmaskt, p['e2_w1'].T, p['e2_w2'].T, evecst,
                   p['fc_w1'].T, p['fc_w2'].T, p['fc_w3'].T, fvecst, m128)
    return logits
